# Initial kernel scaffold; baseline (speedup 1.0000x reference)
import jax
import jax.numpy as jnp
from jax.experimental import pallas as pl


def kernel(descriptors, gradients, grad_index, positions, Z, box, W0, b0, W1, b1):
    raise NotImplementedError("write your pallas kernel here")


# speedup vs baseline: 23.5850x; 23.5850x over previous
import jax
import jax.numpy as jnp
from jax.experimental import pallas as pl


_BLOCK = 2048


def _body(desc_ref, z_ref, w0_ref, b0_ref, w1_ref, out_ref):
    i = pl.program_id(0)
    q_bf = desc_ref[...].astype(jnp.bfloat16)
    z_row = z_ref[...].reshape(1, -1)
    num_types = w0_ref.shape[0]
    total = jnp.zeros((1, 1), jnp.float32)
    for t in range(num_types):
        a_t = jnp.dot(q_bf, w0_ref[t].astype(jnp.bfloat16),
                      preferred_element_type=jnp.float32)
        a_t = a_t + b0_ref[t][None, :]
        r_t = jnp.sum(jnp.tanh(a_t) * w1_ref[t][None, :],
                      axis=1, keepdims=True)
        m_t = (z_row == t).astype(jnp.float32)
        total = total + jnp.dot(m_t, r_t,
                                preferred_element_type=jnp.float32,
                                precision=jax.lax.Precision.HIGHEST)

    @pl.when(i == 0)
    def _():
        out_ref[...] = jnp.zeros_like(out_ref)

    out_ref[...] += total


def kernel(descriptors, gradients, grad_index, positions, Z, box, W0, b0, W1, b1):
    n, d = descriptors.shape
    t, _, h = W0.shape
    block = min(_BLOCK, n)
    nb = n // block
    z3 = Z.astype(jnp.int32).reshape(nb, 1, block)

    out = pl.pallas_call(
        _body,
        grid=(nb,),
        in_specs=[
            pl.BlockSpec((block, d), lambda i: (i, 0)),
            pl.BlockSpec((1, 1, block), lambda i: (i, 0, 0)),
            pl.BlockSpec((t, d, h), lambda i: (0, 0, 0)),
            pl.BlockSpec((t, h), lambda i: (0, 0)),
            pl.BlockSpec((t, h), lambda i: (0, 0)),
        ],
        out_specs=pl.BlockSpec((1, 1), lambda i: (0, 0)),
        out_shape=jax.ShapeDtypeStruct((1, 1), jnp.float32),
    )(descriptors, z3, W0, b0, W1)
    return -(out[0, 0] + n * b1)
